# baseline (speedup 1.0000x reference)
import jax
import jax.numpy as jnp
from jax.experimental import pallas as pl
from jax.experimental.pallas import tpu as pltpu

_MAX_DET = 300
_SCORE_THR = 0.05
_IOU_THR = 0.5
_ROWS = 160
_LANES = 128
_NPAD = _ROWS * _LANES
_NSLOT = 1024
_NEG = float("-inf")
_BIGF = 3.0e7
_UNROLL = 8


def _nms_kernel(cls_ref, bx_ref, kx1_ref, ky1_ref, kx2_ref, ky2_ref,
                ksc_ref, klb_ref, ka_ref):
    C = cls_ref.shape[0]
    shp = (_ROWS, _LANES)

    def cls_body(c, carry):
        mx, am = carry
        x = cls_ref[c]
        upd = x > mx
        mx = jnp.where(upd, x, mx)
        am = jnp.where(upd, jnp.broadcast_to(c.astype(jnp.float32), shp), am)
        return mx, am

    scores, labf = jax.lax.fori_loop(
        0, C, cls_body,
        (jnp.full(shp, _NEG, jnp.float32), jnp.zeros(shp, jnp.float32)))

    x1 = bx_ref[0]
    y1 = bx_ref[1]
    x2 = bx_ref[2]
    y2 = bx_ref[3]

    row_i = jax.lax.broadcasted_iota(jnp.int32, shp, 0)
    lane_i = jax.lax.broadcasted_iota(jnp.int32, shp, 1)
    lin_f = (row_i * _LANES + lane_i).astype(jnp.float32)
    ksub = jax.lax.broadcasted_iota(jnp.int32, (8, _LANES), 0)
    klane = jax.lax.broadcasted_iota(jnp.int32, (8, _LANES), 1)
    kiota = ksub * _LANES + klane

    kx1_ref[...] = jnp.full((8, _LANES), -1.0, jnp.float32)
    ky1_ref[...] = jnp.full((8, _LANES), -1.0, jnp.float32)
    kx2_ref[...] = jnp.full((8, _LANES), -1.0, jnp.float32)
    ky2_ref[...] = jnp.full((8, _LANES), -1.0, jnp.float32)
    ksc_ref[...] = jnp.full((8, _LANES), -1.0, jnp.float32)
    klb_ref[...] = jnp.full((8, _LANES), -1, jnp.int32)
    ka_ref[...] = jnp.zeros((8, _LANES), jnp.float32)

    ones_mat = jnp.ones((_LANES, _LANES), jnp.float32)

    cur0 = jnp.where(scores > _SCORE_THR, scores, _NEG)

    def pop(cur, cnt):
        m = jnp.max(cur, axis=(0, 1), keepdims=True)
        alive = m > _NEG
        linv = jnp.min(jnp.where(cur == m, lin_f, _BIGF),
                       axis=(0, 1), keepdims=True)
        sel = lin_f == linv
        cur = jnp.where(sel, _NEG, cur)

        fsel = sel.astype(jnp.float32)
        px1 = jnp.sum(fsel * x1, axis=0, keepdims=True)
        py1 = jnp.sum(fsel * y1, axis=0, keepdims=True)
        px2 = jnp.sum(fsel * x2, axis=0, keepdims=True)
        py2 = jnp.sum(fsel * y2, axis=0, keepdims=True)
        plb = jnp.sum(fsel * labf, axis=0, keepdims=True)
        pk = jnp.concatenate([px1, py1, px2, py2, plb], axis=0)
        ex = jax.lax.dot_general(
            pk, ones_mat, (((1,), (0,)), ((), ())),
            precision=jax.lax.Precision.HIGHEST,
            preferred_element_type=jnp.float32)
        x1i = ex[0:1, :]
        y1i = ex[1:2, :]
        x2i = ex[2:3, :]
        y2i = ex[3:4, :]
        li = ex[4:5, :]
        ai = (x2i - x1i) * (y2i - y1i)

        kx1 = kx1_ref[...]
        ky1 = ky1_ref[...]
        kx2 = kx2_ref[...]
        ky2 = ky2_ref[...]
        ka = ka_ref[...]
        xx1 = jnp.maximum(kx1, x1i)
        yy1 = jnp.maximum(ky1, y1i)
        xx2 = jnp.minimum(kx2, x2i)
        yy2 = jnp.minimum(ky2, y2i)
        inter = jnp.maximum(0.0, xx2 - xx1) * jnp.maximum(0.0, yy2 - yy1)
        iou = inter / (ka + ai - inter + 1e-8)
        gtf = (iou > _IOU_THR).astype(jnp.float32)
        scnt = jax.lax.dot_general(
            gtf, ones_mat, (((1,), (0,)), ((), ())),
            precision=jax.lax.Precision.HIGHEST,
            preferred_element_type=jnp.float32)
        supp = jnp.sum(scnt, axis=0, keepdims=True) > 0.0
        accept = alive & jnp.logical_not(supp)

        put = accept & (kiota == cnt)
        kx1_ref[...] = jnp.where(put, x1i, kx1)
        ky1_ref[...] = jnp.where(put, y1i, ky1)
        kx2_ref[...] = jnp.where(put, x2i, kx2)
        ky2_ref[...] = jnp.where(put, y2i, ky2)
        ka_ref[...] = jnp.where(put, ai, ka)
        ksc_ref[...] = jnp.where(put, m, ksc_ref[...])
        klb_ref[...] = jnp.where(put, li.astype(jnp.int32), klb_ref[...])
        cnt = cnt + accept.astype(jnp.int32)
        return cur, cnt, m

    def cond(carry):
        return carry[2]

    def body(carry):
        cur, cnt, _ = carry
        m = None
        for _ in range(_UNROLL):
            cur, cnt, m = pop(cur, cnt)
        go = (jnp.max(m) > _NEG) & (jnp.max(cnt) < _MAX_DET)
        return cur, cnt, go

    jax.lax.while_loop(
        cond, body,
        (cur0, jnp.zeros((1, _LANES), jnp.int32), jnp.bool_(True)))


@jax.jit
def kernel(boxes, classification):
    B, N, C = classification.shape
    pad = _NPAD - N
    cls_t = jnp.pad(classification, ((0, 0), (0, pad), (0, 0)),
                    constant_values=-1.0)
    cls_t = cls_t.transpose(0, 2, 1).reshape(B, C, _ROWS, _LANES)
    bx = jnp.pad(boxes, ((0, 0), (0, pad), (0, 0)))
    bx = bx.transpose(0, 2, 1).reshape(B, 4, _ROWS, _LANES)

    kshape = jax.ShapeDtypeStruct((B, 8, _LANES), jnp.float32)
    kspec = pl.BlockSpec((None, 8, _LANES), lambda b: (b, 0, 0))
    kx1, ky1, kx2, ky2, ksc, klb = pl.pallas_call(
        _nms_kernel,
        grid=(B,),
        in_specs=[
            pl.BlockSpec((None, C, _ROWS, _LANES), lambda b: (b, 0, 0, 0)),
            pl.BlockSpec((None, 4, _ROWS, _LANES), lambda b: (b, 0, 0, 0)),
        ],
        out_specs=[kspec] * 6,
        out_shape=[kshape, kshape, kshape, kshape, kshape,
                   jax.ShapeDtypeStruct((B, 8, _LANES), jnp.int32)],
        scratch_shapes=[
            pltpu.VMEM((8, _LANES), jnp.float32),
        ],
    )(cls_t, bx)
    ob = jnp.stack([a.reshape(B, _NSLOT)[:, :_MAX_DET]
                    for a in (kx1, ky1, kx2, ky2)], axis=-1)
    return (ob, ksc.reshape(B, _NSLOT)[:, :_MAX_DET],
            klb.reshape(B, _NSLOT)[:, :_MAX_DET])


# speedup vs baseline: 1.1757x; 1.1757x over previous
import jax
import jax.numpy as jnp
from jax.experimental import pallas as pl
from jax.experimental.pallas import tpu as pltpu

_MAX_DET = 300
_SCORE_THR = 0.05
_IOU_THR = 0.5
_ROWS = 160
_LANES = 128
_NPAD = _ROWS * _LANES
_NSLOT = 1024
_NEG = float("-inf")
_BIGF = 3.0e7
_UNROLL = 8


def _nms_kernel(cls_ref, bx_ref, kx1_ref, ky1_ref, kx2_ref, ky2_ref,
                ksc_ref, klb_ref, ka_ref):
    C = cls_ref.shape[0]
    shp = (_ROWS, _LANES)

    def cls_body(c, carry):
        mx, am = carry
        x = cls_ref[c]
        upd = x > mx
        mx = jnp.where(upd, x, mx)
        am = jnp.where(upd, jnp.broadcast_to(c.astype(jnp.float32), shp), am)
        return mx, am

    scores, labf = jax.lax.fori_loop(
        0, C, cls_body,
        (jnp.full(shp, _NEG, jnp.float32), jnp.zeros(shp, jnp.float32)))

    x1 = bx_ref[0]
    y1 = bx_ref[1]
    x2 = bx_ref[2]
    y2 = bx_ref[3]

    row_i = jax.lax.broadcasted_iota(jnp.int32, shp, 0)
    lane_i = jax.lax.broadcasted_iota(jnp.int32, shp, 1)
    lin_f = (row_i * _LANES + lane_i).astype(jnp.float32)
    ksub = jax.lax.broadcasted_iota(jnp.int32, (8, _LANES), 0)
    klane = jax.lax.broadcasted_iota(jnp.int32, (8, _LANES), 1)
    kiota = ksub * _LANES + klane

    kx1_ref[...] = jnp.full((8, _LANES), -1.0, jnp.float32)
    ky1_ref[...] = jnp.full((8, _LANES), -1.0, jnp.float32)
    kx2_ref[...] = jnp.full((8, _LANES), -1.0, jnp.float32)
    ky2_ref[...] = jnp.full((8, _LANES), -1.0, jnp.float32)
    ksc_ref[...] = jnp.full((8, _LANES), -1.0, jnp.float32)
    klb_ref[...] = jnp.full((8, _LANES), -1, jnp.int32)
    ka_ref[...] = jnp.zeros((8, _LANES), jnp.float32)

    ones_mat = jnp.ones((_LANES, _LANES), jnp.float32)

    cur0 = jnp.where(scores > _SCORE_THR, scores, _NEG)

    def pop(cur, cnt):
        m_row = jnp.max(cur, axis=0, keepdims=True)
        m = jnp.max(m_row, axis=1, keepdims=True)
        alive = m > _NEG
        colarg = jnp.min(jnp.where(cur == m, lin_f, _BIGF),
                         axis=0, keepdims=True)
        linv = jnp.min(colarg, axis=1, keepdims=True)
        sel = lin_f == linv
        cur = jnp.where(sel, _NEG, cur)

        fsel = sel.astype(jnp.float32)
        px1 = jnp.sum(fsel * x1, axis=0, keepdims=True)
        py1 = jnp.sum(fsel * y1, axis=0, keepdims=True)
        px2 = jnp.sum(fsel * x2, axis=0, keepdims=True)
        py2 = jnp.sum(fsel * y2, axis=0, keepdims=True)
        plb = jnp.sum(fsel * labf, axis=0, keepdims=True)
        pk = jnp.concatenate([px1, py1, px2, py2, plb], axis=0)
        ex = jax.lax.dot_general(
            pk, ones_mat, (((1,), (0,)), ((), ())),
            precision=jax.lax.Precision.HIGHEST,
            preferred_element_type=jnp.float32)
        x1i = ex[0:1, :]
        y1i = ex[1:2, :]
        x2i = ex[2:3, :]
        y2i = ex[3:4, :]
        li = ex[4:5, :]
        ai = (x2i - x1i) * (y2i - y1i)

        kx1 = kx1_ref[...]
        ky1 = ky1_ref[...]
        kx2 = kx2_ref[...]
        ky2 = ky2_ref[...]
        ka = ka_ref[...]
        xx1 = jnp.maximum(kx1, x1i)
        yy1 = jnp.maximum(ky1, y1i)
        xx2 = jnp.minimum(kx2, x2i)
        yy2 = jnp.minimum(ky2, y2i)
        inter = jnp.maximum(0.0, xx2 - xx1) * jnp.maximum(0.0, yy2 - yy1)
        iou = inter / (ka + ai - inter + 1e-8)
        gtf = (iou > _IOU_THR).astype(jnp.float32)
        scnt = jax.lax.dot_general(
            gtf, ones_mat, (((1,), (0,)), ((), ())),
            precision=jax.lax.Precision.HIGHEST,
            preferred_element_type=jnp.float32)
        supp = jnp.sum(scnt, axis=0, keepdims=True) > 0.0
        accept = alive & jnp.logical_not(supp)

        put = accept & (kiota == cnt)
        kx1_ref[...] = jnp.where(put, x1i, kx1)
        ky1_ref[...] = jnp.where(put, y1i, ky1)
        kx2_ref[...] = jnp.where(put, x2i, kx2)
        ky2_ref[...] = jnp.where(put, y2i, ky2)
        ka_ref[...] = jnp.where(put, ai, ka)
        ksc_ref[...] = jnp.where(put, m, ksc_ref[...])
        klb_ref[...] = jnp.where(put, li.astype(jnp.int32), klb_ref[...])
        cnt = cnt + accept.astype(jnp.int32)
        return cur, cnt, m

    def cond(carry):
        return carry[2]

    def body(carry):
        cur, cnt, _ = carry
        m = None
        for _ in range(_UNROLL):
            cur, cnt, m = pop(cur, cnt)
        go = (jnp.max(m) > _NEG) & (jnp.max(cnt) < _MAX_DET)
        return cur, cnt, go

    jax.lax.while_loop(
        cond, body,
        (cur0, jnp.zeros((1, _LANES), jnp.int32), jnp.bool_(True)))


@jax.jit
def kernel(boxes, classification):
    B, N, C = classification.shape
    pad = _NPAD - N
    cls_t = jnp.pad(classification, ((0, 0), (0, pad), (0, 0)),
                    constant_values=-1.0)
    cls_t = cls_t.transpose(0, 2, 1).reshape(B, C, _ROWS, _LANES)
    bx = jnp.pad(boxes, ((0, 0), (0, pad), (0, 0)))
    bx = bx.transpose(0, 2, 1).reshape(B, 4, _ROWS, _LANES)

    kshape = jax.ShapeDtypeStruct((B, 8, _LANES), jnp.float32)
    kspec = pl.BlockSpec((None, 8, _LANES), lambda b: (b, 0, 0))
    kx1, ky1, kx2, ky2, ksc, klb = pl.pallas_call(
        _nms_kernel,
        grid=(B,),
        in_specs=[
            pl.BlockSpec((None, C, _ROWS, _LANES), lambda b: (b, 0, 0, 0)),
            pl.BlockSpec((None, 4, _ROWS, _LANES), lambda b: (b, 0, 0, 0)),
        ],
        out_specs=[kspec] * 6,
        out_shape=[kshape, kshape, kshape, kshape, kshape,
                   jax.ShapeDtypeStruct((B, 8, _LANES), jnp.int32)],
        scratch_shapes=[
            pltpu.VMEM((8, _LANES), jnp.float32),
        ],
    )(cls_t, bx)
    ob = jnp.stack([a.reshape(B, _NSLOT)[:, :_MAX_DET]
                    for a in (kx1, ky1, kx2, ky2)], axis=-1)
    return (ob, ksc.reshape(B, _NSLOT)[:, :_MAX_DET],
            klb.reshape(B, _NSLOT)[:, :_MAX_DET])
